# Initial kernel scaffold; baseline (speedup 1.0000x reference)
import jax
import jax.numpy as jnp
from jax.experimental import pallas as pl


def kernel(x, edge_index, W1, att_src1, att_dst1, bias1, bn_gamma, bn_beta, bn_mean, bn_var, W2, att_src2, att_dst2, bias2, Wc, bc):
    raise NotImplementedError("write your pallas kernel here")


# speedup vs baseline: 1.0795x; 1.0795x over previous
import functools
import math

import jax
import jax.numpy as jnp
from jax.experimental import pallas as pl

N = 50000
E = 800000
IN_CH = 13
H1, D1 = 4, 64
H2, D2 = 2, 32

BN = 1024
NPAD = ((N + BN - 1) // BN) * BN


def _proj_kernel(x_ref, w_ref, atts_ref, attd_ref, h_ref, as_ref, ad_ref, *, heads, dim):
    h = jnp.dot(x_ref[...], w_ref[...], preferred_element_type=jnp.float32)
    h_ref[...] = h
    hv = h.reshape(h.shape[0], heads, dim)
    as_ref[...] = jnp.sum(hv * atts_ref[...][None], axis=-1)
    ad_ref[...] = jnp.sum(hv * attd_ref[...][None], axis=-1)


def _project(x, W, att_s, att_d, heads, dim):
    n = x.shape[0]
    npad = ((n + BN - 1) // BN) * BN
    xp = jnp.pad(x, ((0, npad - n), (0, 0)))
    grid = npad // BN
    h, a_s, a_d = pl.pallas_call(
        functools.partial(_proj_kernel, heads=heads, dim=dim),
        grid=(grid,),
        in_specs=[
            pl.BlockSpec((BN, x.shape[1]), lambda i: (i, 0)),
            pl.BlockSpec((x.shape[1], heads * dim), lambda i: (0, 0)),
            pl.BlockSpec((heads, dim), lambda i: (0, 0)),
            pl.BlockSpec((heads, dim), lambda i: (0, 0)),
        ],
        out_specs=[
            pl.BlockSpec((BN, heads * dim), lambda i: (i, 0)),
            pl.BlockSpec((BN, heads), lambda i: (i, 0)),
            pl.BlockSpec((BN, heads), lambda i: (i, 0)),
        ],
        out_shape=[
            jax.ShapeDtypeStruct((npad, heads * dim), jnp.float32),
            jax.ShapeDtypeStruct((npad, heads), jnp.float32),
            jax.ShapeDtypeStruct((npad, heads), jnp.float32),
        ],
    )(xp, W, att_s, att_d)
    return h[:n], a_s[:n], a_d[:n]


def _gat_layer(x, src, dst, W, att_s, att_d, bias, heads, dim, concat, n):
    h, a_s, a_d = _project(x, W, att_s, att_d, heads, dim)
    alpha = a_s[src] + a_d[dst]
    alpha = jnp.where(alpha > 0, alpha, 0.2 * alpha)
    alpha = jnp.exp(alpha)
    denom = jax.ops.segment_sum(alpha, dst, num_segments=n)
    hv = h.reshape(n, heads, dim)
    msg = hv[src] * alpha[:, :, None]
    out = jax.ops.segment_sum(msg, dst, num_segments=n)
    out = out / (denom[:, :, None] + 1e-16)
    if concat:
        out = out.reshape(n, heads * dim)
    else:
        out = out.mean(axis=1)
    return out + bias


def kernel(x, edge_index, W1, att_src1, att_dst1, bias1, bn_gamma, bn_beta,
           bn_mean, bn_var, W2, att_src2, att_dst2, bias2, Wc, bc):
    n = x.shape[0]
    loop = jnp.arange(n, dtype=edge_index.dtype)
    src = jnp.concatenate([edge_index[0], loop])
    dst = jnp.concatenate([edge_index[1], loop])
    h1 = _gat_layer(x, src, dst, W1, att_src1, att_dst1, bias1, H1, D1, True, n)
    h1 = (h1 - bn_mean) / jnp.sqrt(bn_var + 1e-5) * bn_gamma + bn_beta
    h1 = jax.nn.elu(h1)
    h2 = _gat_layer(h1, src, dst, W2, att_src2, att_dst2, bias2, H2, D2, False, n)
    h2 = jax.nn.elu(h2)
    return h2 @ Wc + bc
